# baseline (speedup 1.0000x reference)
import functools

import jax
import jax.numpy as jnp
from jax import lax
from jax.experimental import pallas as pl
import jax.experimental.pallas.tpu as pltpu
from jax.experimental.pallas import tpu_sc as plsc

_B, _H, _KV, _Q, _DH = 8, 16, 2048, 16, 128
_BH = _B * _H

_RC = 2
_NT = _BH // _RC
_NBUF = 8
_L = 4

_NC, _NS = 2, 16
_NW = _NC * _NS
_RW = _BH // _NW
_CH = 128
_NCH = _KV // _CH
_SCU = _RW * _NCH
_SNB = 6
_SL = 3


def _tc_pipeline(pk_ref, ks_ref, ok_ref, kbuf, kin, kout):
    def in_copies(i, s):
        rows = pl.ds(i * _RC, _RC)
        return [
            pltpu.make_async_copy(pk_ref.at[rows], kbuf.at[s, :, pl.ds(0, _KV)], kin.at[s]),
            pltpu.make_async_copy(ks_ref.at[rows], kbuf.at[s, :, pl.ds(_KV, _Q)], kin.at[s]),
        ]

    def out_copies(i, s):
        rows = pl.ds(i * _RC, _RC)
        return [pltpu.make_async_copy(kbuf.at[s], ok_ref.at[rows], kout.at[s])]

    for j in range(_L):
        for c in in_copies(j, j % _NBUF):
            c.start()
    for i in range(_NT):
        s = i % _NBUF
        nxt = i + _L
        if nxt < _NT:
            if nxt - _NBUF >= 0:
                for c in out_copies(nxt - _NBUF, nxt % _NBUF):
                    c.wait()
            for c in in_copies(nxt, nxt % _NBUF):
                c.start()
        for c in in_copies(i, s):
            c.wait()
        for c in out_copies(i, s):
            c.start()
    for j in range(_NT - _NBUF, _NT):
        for c in out_copies(j, j % _NBUF):
            c.wait()


def _sc_copy(pv, vs, ov, vbuf, sbuf, rsem, wsem):
    wid = lax.axis_index("s") * _NC + lax.axis_index("c")
    base = wid * _RW

    def rd(u, b):
        row = base + u // _NCH
        sl = pl.ds((u % _NCH) * _CH, _CH)
        return pltpu.make_async_copy(pv.at[row, sl], vbuf.at[b], rsem.at[b])

    def wr(u, b):
        row = base + u // _NCH
        sl = pl.ds((u % _NCH) * _CH, _CH)
        return pltpu.make_async_copy(vbuf.at[b], ov.at[row, sl], wsem.at[b])

    h2h = []
    for j in range(_RW):
        row = base + j
        h2h.append(pltpu.make_async_copy(
            pv.at[row], ov.at[row, pl.ds(0, _KV)], rsem.at[j % _SNB]))
    for c in h2h:
        c.start()
    for c in h2h:
        c.wait()
    tail = pl.ds(_KV, _Q)
    for j in range(_RW):
        row = base + j
        pltpu.sync_copy(vs.at[row], sbuf)
        pltpu.sync_copy(sbuf, ov.at[row, tail])


def kernel(past_key, past_value, key_states, value_states, layer_idx):
    pk = past_key.reshape(_BH, _KV, _DH)
    pv = past_value.reshape(_BH, _KV, _DH)
    ks = key_states.reshape(_BH, _Q, _DH)
    vs = value_states.reshape(_BH, _Q, _DH)

    out_t = jax.ShapeDtypeStruct((_BH, _KV + _Q, _DH), jnp.float32)
    hbm_spec = pl.BlockSpec(memory_space=pltpu.MemorySpace.HBM)

    ok = pl.pallas_call(
        _tc_pipeline,
        in_specs=[hbm_spec, hbm_spec],
        out_specs=hbm_spec,
        out_shape=out_t,
        scratch_shapes=[
            pltpu.MemorySpace.VMEM((_NBUF, _RC, _KV + _Q, _DH), jnp.float32),
            pltpu.SemaphoreType.DMA((_NBUF,)),
            pltpu.SemaphoreType.DMA((_NBUF,)),
        ],
    )(pk, ks)

    sc_mesh = plsc.VectorSubcoreMesh(
        core_axis_name="c", subcore_axis_name="s",
        num_cores=_NC, num_subcores=_NS)
    sc_copy = functools.partial(
        pl.kernel, mesh=sc_mesh,
        out_type=out_t,
        scratch_types=[
            pltpu.VMEM((_SNB, _CH, _DH), jnp.float32),
            pltpu.VMEM((_Q, _DH), jnp.float32),
            pltpu.SemaphoreType.DMA((_SNB,)),
            pltpu.SemaphoreType.DMA((_SNB,)),
        ],
    )(_sc_copy)
    ov = sc_copy(pv, vs)

    ok = ok.reshape(_B, _H, _KV + _Q, _DH)
    ov = ov.reshape(_B, _H, _KV + _Q, _DH)
    return (ok, ov)


# speedup vs baseline: 24.4210x; 24.4210x over previous
import jax
import jax.numpy as jnp
from jax.experimental import pallas as pl
import jax.experimental.pallas.tpu as pltpu

_B, _H, _KV, _Q, _DH = 8, 16, 2048, 16, 128
_BH = _B * _H
_N = _BH
_NBUF = 8
_L = 4
_HK = _KV // 2
_HO = (_KV + _Q) // 2


def _dma_pipeline(pk_ref, pv_ref, ks_ref, vs_ref, ok_ref, ov_ref,
                  kbuf, vbuf,
                  kin1, kin2, vin1, vin2,
                  kout1, kout2, vout1, vout2):
    def in_copies(i, s):
        return [
            pltpu.make_async_copy(pk_ref.at[i, pl.ds(0, _HK)],
                                  kbuf.at[s, pl.ds(0, _HK)], kin1.at[s]),
            pltpu.make_async_copy(pk_ref.at[i, pl.ds(_HK, _HK)],
                                  kbuf.at[s, pl.ds(_HK, _HK)], kin2.at[s]),
            pltpu.make_async_copy(ks_ref.at[i],
                                  kbuf.at[s, pl.ds(_KV, _Q)], kin1.at[s]),
            pltpu.make_async_copy(pv_ref.at[i, pl.ds(0, _HK)],
                                  vbuf.at[s, pl.ds(0, _HK)], vin1.at[s]),
            pltpu.make_async_copy(pv_ref.at[i, pl.ds(_HK, _HK)],
                                  vbuf.at[s, pl.ds(_HK, _HK)], vin2.at[s]),
            pltpu.make_async_copy(vs_ref.at[i],
                                  vbuf.at[s, pl.ds(_KV, _Q)], vin1.at[s]),
        ]

    def out_copies(i, s):
        return [
            pltpu.make_async_copy(kbuf.at[s, pl.ds(0, _HO)],
                                  ok_ref.at[i, pl.ds(0, _HO)], kout1.at[s]),
            pltpu.make_async_copy(kbuf.at[s, pl.ds(_HO, _HO)],
                                  ok_ref.at[i, pl.ds(_HO, _HO)], kout2.at[s]),
            pltpu.make_async_copy(vbuf.at[s, pl.ds(0, _HO)],
                                  ov_ref.at[i, pl.ds(0, _HO)], vout1.at[s]),
            pltpu.make_async_copy(vbuf.at[s, pl.ds(_HO, _HO)],
                                  ov_ref.at[i, pl.ds(_HO, _HO)], vout2.at[s]),
        ]

    for j in range(_L):
        for c in in_copies(j, j % _NBUF):
            c.start()
    for i in range(_N):
        s = i % _NBUF
        nxt = i + _L
        if nxt < _N:
            if nxt - _NBUF >= 0:
                for c in out_copies(nxt - _NBUF, nxt % _NBUF):
                    c.wait()
            for c in in_copies(nxt, nxt % _NBUF):
                c.start()
        for c in in_copies(i, s):
            c.wait()
        for c in out_copies(i, s):
            c.start()
    for j in range(_N - _NBUF, _N):
        for c in out_copies(j, j % _NBUF):
            c.wait()


def kernel(past_key, past_value, key_states, value_states, layer_idx):
    pk = past_key.reshape(_BH, _KV, _DH)
    pv = past_value.reshape(_BH, _KV, _DH)
    ks = key_states.reshape(_BH, _Q, _DH)
    vs = value_states.reshape(_BH, _Q, _DH)

    hbm_spec = pl.BlockSpec(memory_space=pltpu.MemorySpace.HBM)
    out_shape = jax.ShapeDtypeStruct((_BH, _KV + _Q, _DH), jnp.float32)

    ok, ov = pl.pallas_call(
        _dma_pipeline,
        in_specs=[hbm_spec] * 4,
        out_specs=[hbm_spec, hbm_spec],
        out_shape=[out_shape, out_shape],
        scratch_shapes=[
            pltpu.MemorySpace.VMEM((_NBUF, _KV + _Q, _DH), jnp.float32),
            pltpu.MemorySpace.VMEM((_NBUF, _KV + _Q, _DH), jnp.float32),
        ] + [pltpu.SemaphoreType.DMA((_NBUF,))] * 8,
    )(pk, pv, ks, vs)

    ok = ok.reshape(_B, _H, _KV + _Q, _DH)
    ov = ov.reshape(_B, _H, _KV + _Q, _DH)
    return (ok, ov)
